# baseline (device time: 322454 ns/iter reference)
import jax
import jax.numpy as jnp
from jax import lax
from jax.experimental import pallas as pl
from jax.experimental.pallas import tpu as pltpu

N_DEV = 4


def _gelu(z):
    return 0.5 * z * (1.0 + jnp.tanh(0.7978845608 * (z + 0.044715 * z * z * z)))


def kernel(A, B):
    m, k = A.shape
    _, n = B.shape
    m_chunk = m // N_DEV

    def body(a_ref, b_ref, out_ref, comm_ref, send_sems, recv_sems):
        my = lax.axis_index("i")
        left = (my - 1) % N_DEV
        right = (my + 1) % N_DEV

        barrier_sem = pltpu.get_barrier_semaphore()
        for nbr in (left, right):
            pl.semaphore_signal(
                barrier_sem, inc=1,
                device_id=(nbr,), device_id_type=pl.DeviceIdType.MESH,
            )
        pl.semaphore_wait(barrier_sem, 2)

        out_ref[:, :] = jnp.dot(
            a_ref[:, :], b_ref[:, :], preferred_element_type=jnp.float32
        )

        for h in range(N_DEV - 1):
            send_c = (my - h) % N_DEV
            recv_c = (my - h - 1) % N_DEV
            slot = h % 2
            rdma = pltpu.make_async_remote_copy(
                src_ref=out_ref.at[pl.ds(send_c * m_chunk, m_chunk), :],
                dst_ref=comm_ref.at[slot],
                send_sem=send_sems.at[h],
                recv_sem=recv_sems.at[h],
                device_id=(right,),
                device_id_type=pl.DeviceIdType.MESH,
            )
            rdma.start()
            rdma.wait()
            rows = pl.ds(recv_c * m_chunk, m_chunk)
            out_ref[rows, :] = out_ref[rows, :] + comm_ref[slot, :, :]

        own = (my + 1) % N_DEV
        rows = pl.ds(own * m_chunk, m_chunk)
        out_ref[rows, :] = _gelu(out_ref[rows, :])

        for h in range(N_DEV - 1):
            send_c = (my + 1 - h) % N_DEV
            recv_c = (my - h) % N_DEV
            slot = (h + 1) % 2
            rdma = pltpu.make_async_remote_copy(
                src_ref=out_ref.at[pl.ds(send_c * m_chunk, m_chunk), :],
                dst_ref=comm_ref.at[slot],
                send_sem=send_sems.at[N_DEV - 1 + h],
                recv_sem=recv_sems.at[N_DEV - 1 + h],
                device_id=(right,),
                device_id_type=pl.DeviceIdType.MESH,
            )
            rdma.start()
            rdma.wait()
            out_ref[pl.ds(recv_c * m_chunk, m_chunk), :] = comm_ref[slot, :, :]

    return pl.pallas_call(
        body,
        out_shape=jax.ShapeDtypeStruct((m, n), jnp.float32),
        in_specs=[
            pl.BlockSpec(memory_space=pltpu.VMEM),
            pl.BlockSpec(memory_space=pltpu.VMEM),
        ],
        out_specs=pl.BlockSpec(memory_space=pltpu.VMEM),
        scratch_shapes=[
            pltpu.VMEM((2, m_chunk, n), jnp.float32),
            pltpu.SemaphoreType.DMA((2 * (N_DEV - 1),)),
            pltpu.SemaphoreType.DMA((2 * (N_DEV - 1),)),
        ],
        compiler_params=pltpu.CompilerParams(collective_id=0),
    )(A, B)


# device time: 111060 ns/iter; 2.9034x vs baseline; 2.9034x over previous
import jax
import jax.numpy as jnp
from jax import lax
from jax.experimental import pallas as pl
from jax.experimental.pallas import tpu as pltpu

N_DEV = 4
CW, CCW = 0, 1


def _gelu(z):
    return 0.5 * z * (1.0 + jnp.tanh(0.7978845608 * (z + 0.044715 * z * z * z)))


def kernel(A, B):
    m, k = A.shape
    _, n = B.shape
    half = m // 2
    mc = half // N_DEV

    def body(a_ref, b_ref, out_ref, acc_ref, b16_ref, comm_cw, comm_ccw,
             send_sems, recv_sems):
        my = lax.axis_index("i")
        left = (my - 1) % N_DEV
        right = (my + 1) % N_DEV

        barrier_sem = pltpu.get_barrier_semaphore()
        for nbr in (left, right):
            pl.semaphore_signal(
                barrier_sem, inc=1,
                device_id=(nbr,), device_id_type=pl.DeviceIdType.MESH,
            )
        pl.semaphore_wait(barrier_sem, 2)

        b16_ref[:, :] = b_ref[:, :].astype(jnp.bfloat16)

        def top_rows(c):
            return pl.ds((c % N_DEV) * mc, mc)

        def bot_rows(c):
            return pl.ds(half + (c % N_DEV) * mc, mc)

        def compute_chunk(rows):
            acc_ref[rows, :] = jnp.dot(
                a_ref[rows, :].astype(jnp.bfloat16), b16_ref[:, :],
                preferred_element_type=jnp.float32,
            ).astype(jnp.bfloat16)

        compute_chunk(top_rows(my))
        compute_chunk(bot_rows(my))
        for h in range(N_DEV - 1):
            slot = h % 2
            cw = pltpu.make_async_remote_copy(
                src_ref=acc_ref.at[top_rows(my - h), :],
                dst_ref=comm_cw.at[slot],
                send_sem=send_sems.at[CW, h],
                recv_sem=recv_sems.at[CW, h],
                device_id=(right,),
                device_id_type=pl.DeviceIdType.MESH,
            )
            ccw = pltpu.make_async_remote_copy(
                src_ref=acc_ref.at[bot_rows(my + h), :],
                dst_ref=comm_ccw.at[slot],
                send_sem=send_sems.at[CCW, h],
                recv_sem=recv_sems.at[CCW, h],
                device_id=(left,),
                device_id_type=pl.DeviceIdType.MESH,
            )
            cw.start()
            ccw.start()
            compute_chunk(top_rows(my - h - 1))
            compute_chunk(bot_rows(my + h + 1))
            cw.wait()
            ccw.wait()
            rt = top_rows(my - h - 1)
            acc_ref[rt, :] = acc_ref[rt, :] + comm_cw[slot, :, :]
            rb = bot_rows(my + h + 1)
            acc_ref[rb, :] = acc_ref[rb, :] + comm_ccw[slot, :, :]

        for rows in (top_rows(my + 1), bot_rows(my - 1)):
            g = _gelu(acc_ref[rows, :].astype(jnp.float32))
            out_ref[rows, :] = g
            acc_ref[rows, :] = g.astype(jnp.bfloat16)

        for h in range(N_DEV - 1):
            cw = pltpu.make_async_remote_copy(
                src_ref=acc_ref.at[top_rows(my + 1 - h), :],
                dst_ref=acc_ref.at[top_rows(my + 1 - h), :],
                send_sem=send_sems.at[CW, N_DEV - 1 + h],
                recv_sem=recv_sems.at[CW, N_DEV - 1 + h],
                device_id=(right,),
                device_id_type=pl.DeviceIdType.MESH,
            )
            ccw = pltpu.make_async_remote_copy(
                src_ref=acc_ref.at[bot_rows(my - 1 + h), :],
                dst_ref=acc_ref.at[bot_rows(my - 1 + h), :],
                send_sem=send_sems.at[CCW, N_DEV - 1 + h],
                recv_sem=recv_sems.at[CCW, N_DEV - 1 + h],
                device_id=(left,),
                device_id_type=pl.DeviceIdType.MESH,
            )
            cw.start()
            ccw.start()
            cw.wait()
            ccw.wait()
            rt = top_rows(my - h)
            out_ref[rt, :] = acc_ref[rt, :].astype(jnp.float32)
            rb = bot_rows(my + h)
            out_ref[rb, :] = acc_ref[rb, :].astype(jnp.float32)

    return pl.pallas_call(
        body,
        out_shape=jax.ShapeDtypeStruct((m, n), jnp.float32),
        in_specs=[
            pl.BlockSpec(memory_space=pltpu.VMEM),
            pl.BlockSpec(memory_space=pltpu.VMEM),
        ],
        out_specs=pl.BlockSpec(memory_space=pltpu.VMEM),
        scratch_shapes=[
            pltpu.VMEM((m, n), jnp.bfloat16),
            pltpu.VMEM((k, n), jnp.bfloat16),
            pltpu.VMEM((2, mc, n), jnp.bfloat16),
            pltpu.VMEM((2, mc, n), jnp.bfloat16),
            pltpu.SemaphoreType.DMA((2, 2 * (N_DEV - 1))),
            pltpu.SemaphoreType.DMA((2, 2 * (N_DEV - 1))),
        ],
        compiler_params=pltpu.CompilerParams(
            collective_id=0, vmem_limit_bytes=100 * 1024 * 1024
        ),
    )(A, B)
